# baseline (speedup 1.0000x reference)
import jax
import jax.numpy as jnp
from jax import lax
from jax.experimental import pallas as pl
from jax.experimental.pallas import tpu as pltpu
from jax.experimental.pallas import tpu_sc as plsc

_N = 10000
_E = 320000
_D = 128
_NC, _NS = 2, 16
_HALF = _N // _NC
_ACC = 5120
_RPT = _ACC // _NS
_EPS = _E // _NS
_P = 2048
_NP = _EPS // _P
_PT = _EPS - _NP * _P
_CAPR = _EPS // 16 + 8
_K = 128
_SB = 13


def _sc_body(x_hbm, src_hbm, dst_hbm, out_hbm,
             acc_sh, deg_sh, cfilt, spc, dpc,
             rows0, rows1, sidx0, sidx1, didx0, didx1,
             ones_v, dz, deg_loc, gsem, ssem, dsem):
    c = lax.axis_index("c")
    s = lax.axis_index("s")
    lo = c * _HALF
    rows_b = (rows0, rows1)
    sidx_b = (sidx0, sidx1)
    didx_b = (didx0, didx1)

    for i in range(_K // 16):
        ones_v[pl.ds(i * 16, 16)] = jnp.ones((16,), jnp.float32)
    for i in range(_RPT // 16):
        dz[pl.ds(i * 16, 16)] = jnp.zeros((16,), jnp.float32)

    @pl.loop(0, _K)
    def _zero_rows(r):
        for i in range(_D // 16):
            rows0[r, pl.ds(i * 16, 16)] = jnp.zeros((16,), jnp.float32)

    for t, sz in ((0, _K), (1, _K), (2, _RPT - 2 * _K)):
        pltpu.sync_copy(rows0.at[pl.ds(0, sz), :],
                        acc_sh.at[pl.ds(s * _RPT + t * _K, sz), :])
    pltpu.sync_copy(dz, deg_sh.at[pl.ds(s * _RPT, _RPT)])

    ii = lax.iota(jnp.int32, 16)
    onev = jnp.ones((16,), jnp.int32)
    zerov = jnp.zeros((16,), jnp.int32)

    def _filter_piece(cnts, pbase, pn):
        pltpu.sync_copy(src_hbm.at[pl.ds(pbase, pn)], spc.at[pl.ds(0, pn)])
        pltpu.sync_copy(dst_hbm.at[pl.ds(pbase, pn)], dpc.at[pl.ds(0, pn)])

        @pl.loop(0, pn // 16, init_carry=cnts)
        def _vreg(v, cnts):
            sv = spc[pl.ds(v * 16, 16)]
            dv = dpc[pl.ds(v * 16, 16)]
            m = jnp.logical_and(dv >= lo, dv < lo + _HALF)
            code = lax.shift_left(sv, _SB) + (dv - lo)
            plsc.store_scatter(cfilt, [cnts * 16 + ii], code, mask=m)
            return cnts + jnp.where(m, onev, zerov)

        return _vreg

    cnts = zerov
    ebase = s * _EPS
    for p in range(_NP):
        cnts = _filter_piece(cnts, pl.multiple_of(ebase + p * _P, 8), _P)
    cnts = _filter_piece(cnts, pl.multiple_of(ebase + _NP * _P, 8), _PT)

    maxc = lax.squeeze(lax.slice(cnts, (0,), (1,)), dimensions=(0,))
    for r in range(1, 16):
        maxc = jnp.maximum(maxc, lax.squeeze(
            lax.slice(cnts, (r,), (r + 1,)), dimensions=(0,)))
    maxc_pad = jnp.maximum((maxc + 7) // 8 * 8, 8)

    @pl.loop(0, maxc_pad)
    def _pad_row(r):
        pm = cnts <= r
        padcode = (lax.shift_left((ii * 613 + r * 977) % _N, _SB)
                   + _HALF + (ii + r) % (_ACC - _HALF))
        plsc.store_scatter(cfilt, [r * 16 + ii], padcode, mask=pm)

    nch = maxc_pad // 8

    def _copy_idx(j, b):
        for q in range(_K // 16):
            cv = cfilt[pl.ds(j * _K + q * 16, 16)]
            sidx_b[b][pl.ds(q * 16, 16)] = lax.shift_right_logical(cv, _SB)
            didx_b[b][pl.ds(q * 16, 16)] = lax.bitwise_and(
                cv, jnp.int32(2 ** _SB - 1))

    def _start_gather(b):
        pltpu.async_copy(x_hbm.at[sidx_b[b]], rows_b[b], gsem.at[b])

    def _wait_gather(b):
        pltpu.make_async_copy(x_hbm.at[sidx_b[b]], rows_b[b],
                              gsem.at[b]).wait()

    _copy_idx(0, 0)
    _start_gather(0)
    plsc.subcore_barrier()

    @pl.loop(0, (nch + 1) // 2 * 2, step=2)
    def _chunk(g):
        for b in range(2):
            j = g + b
            nxt = 1 - b

            @pl.when(j < nch)
            def _do_chunk():
                @pl.when(j + 1 < nch)
                def _prefetch():
                    _copy_idx(j + 1, nxt)
                    _start_gather(nxt)

                _wait_gather(b)
                d_acc = pltpu.async_copy(rows_b[b], acc_sh.at[didx_b[b]],
                                         ssem, add=True)
                d_deg = pltpu.async_copy(ones_v, deg_sh.at[didx_b[b]],
                                         dsem, add=True)
                d_acc.wait()
                d_deg.wait()

    plsc.subcore_barrier()

    rbase = s * _RPT
    gband = c * _HALF + s * _RPT
    nlast = _HALF - (_NS - 1) * _RPT
    pltpu.sync_copy(deg_sh.at[pl.ds(rbase, _RPT)], deg_loc)

    for t, sz in ((0, _K), (1, _K), (2, _RPT - 2 * _K)):
        toff = t * _K
        pltpu.sync_copy(acc_sh.at[pl.ds(rbase + toff, sz), :],
                        rows0.at[pl.ds(0, sz), :])

        @pl.loop(0, sz // 16)
        def _div_group(g):
            dvec = deg_loc[pl.ds(toff + g * 16, 16)]
            rvec = 1.0 / jnp.maximum(dvec, 1.0)
            for r in range(16):
                rs = lax.squeeze(lax.slice(rvec, (r,), (r + 1,)),
                                 dimensions=(0,))
                bv = jnp.full((16,), rs, jnp.float32)
                row = g * 16 + r
                for q in range(_D // 16):
                    rows0[row, pl.ds(q * 16, 16)] = (
                        rows0[row, pl.ds(q * 16, 16)] * bv)

        wlast = max(0, min(sz, nlast - toff))

        @pl.when(s < _NS - 1)
        def _wfull():
            pltpu.sync_copy(x_hbm.at[pl.ds(gband + toff, sz), :],
                            rows1.at[pl.ds(0, sz), :])
            pltpu.sync_copy(rows0.at[pl.ds(0, sz), :],
                            out_hbm.at[pl.ds(gband + toff, sz),
                                       pl.ds(_D, _D)])
            pltpu.sync_copy(rows1.at[pl.ds(0, sz), :],
                            out_hbm.at[pl.ds(gband + toff, sz),
                                       pl.ds(0, _D)])

        if wlast > 0:
            @pl.when(s == _NS - 1)
            def _wlast():
                pltpu.sync_copy(x_hbm.at[pl.ds(gband + toff, wlast), :],
                                rows1.at[pl.ds(0, wlast), :])
                pltpu.sync_copy(rows0.at[pl.ds(0, wlast), :],
                                out_hbm.at[pl.ds(gband + toff, wlast),
                                           pl.ds(_D, _D)])
                pltpu.sync_copy(rows1.at[pl.ds(0, wlast), :],
                                out_hbm.at[pl.ds(gband + toff, wlast),
                                           pl.ds(0, _D)])


_sc_call = pl.kernel(
    _sc_body,
    out_type=jax.ShapeDtypeStruct((_N, 2 * _D), jnp.float32),
    mesh=plsc.VectorSubcoreMesh(core_axis_name="c", subcore_axis_name="s"),
    compiler_params=pltpu.CompilerParams(needs_layout_passes=False),
    scratch_types=[
        pltpu.VMEM_SHARED((_ACC, _D), jnp.float32),
        pltpu.VMEM_SHARED((_ACC,), jnp.float32),
        pltpu.VMEM((_CAPR * 16,), jnp.int32),
        pltpu.VMEM((_P,), jnp.int32),
        pltpu.VMEM((_P,), jnp.int32),
        pltpu.VMEM((_K, _D), jnp.float32),
        pltpu.VMEM((_K, _D), jnp.float32),
        pltpu.VMEM((_K,), jnp.int32),
        pltpu.VMEM((_K,), jnp.int32),
        pltpu.VMEM((_K,), jnp.int32),
        pltpu.VMEM((_K,), jnp.int32),
        pltpu.VMEM((_K,), jnp.float32),
        pltpu.VMEM((_RPT,), jnp.float32),
        pltpu.VMEM((_RPT,), jnp.float32),
        pltpu.SemaphoreType.DMA((2,)),
        pltpu.SemaphoreType.DMA,
        pltpu.SemaphoreType.DMA,
    ],
)


def kernel(x, edge_index):
    x = x.astype(jnp.float32)
    ei = edge_index.astype(jnp.int32)
    return _sc_call(x, ei[0], ei[1])


# speedup vs baseline: 1.5278x; 1.5278x over previous
import jax
import jax.numpy as jnp
from jax import lax
from jax.experimental import pallas as pl
from jax.experimental.pallas import tpu as pltpu
from jax.experimental.pallas import tpu_sc as plsc

_N = 10000
_E = 320000
_D = 128
_NC, _NS = 2, 16
_NW = _NC * _NS
_EPT = _E // _NW
_K = 128
_NFULL = _EPT // _K
_TAIL = _EPT - _NFULL * _K
_RPT = 624
_REM = _N - _NS * _RPT
_DEG_PAD = 10240
_DPT = _DEG_PAD // _NS


def _sc_body(x_hbm, edge_hbm,
             acc0_hbm, acc1_hbm, deg0_hbm, deg1_hbm,
             acc_sh, deg_sh, rows0, rows1, sidx0, sidx1, didx0, didx1,
             rows_t, sidx_t, didx_t, ones_v, dz, gsem, isem, ssem, dsem):
    c = lax.axis_index("c")
    s = lax.axis_index("s")
    wid = c * _NS + s
    base = wid * _EPT
    rows_b = (rows0, rows1)
    sidx_b = (sidx0, sidx1)
    didx_b = (didx0, didx1)

    def _start_idx(j, b):
        off = pl.multiple_of(base + j * _K, 8)
        pltpu.async_copy(edge_hbm.at[pl.ds(off, _K)], sidx_b[b],
                         isem.at[b])
        pltpu.async_copy(edge_hbm.at[pl.ds(_E + off, _K)], didx_b[b],
                         isem.at[b])

    def _wait_idx(b):
        pltpu.make_async_copy(edge_hbm.at[pl.ds(0, _K)], sidx_b[b],
                              isem.at[b]).wait()
        pltpu.make_async_copy(edge_hbm.at[pl.ds(0, _K)], didx_b[b],
                              isem.at[b]).wait()

    def _start_gather(b):
        pltpu.async_copy(x_hbm.at[sidx_b[b]], rows_b[b], gsem.at[b])

    def _wait_gather(b):
        pltpu.make_async_copy(x_hbm.at[sidx_b[b]], rows_b[b],
                              gsem.at[b]).wait()

    for i in range(_K // 16):
        ones_v[pl.ds(i * 16, 16)] = jnp.ones((16,), jnp.float32)
    for i in range(_DPT // 16):
        dz[pl.ds(i * 16, 16)] = jnp.zeros((16,), jnp.float32)

    @pl.loop(0, _K)
    def _zero_rows(r):
        for i in range(_D // 16):
            rows0[r, pl.ds(i * 16, 16)] = jnp.zeros((16,), jnp.float32)

    _start_idx(0, 0)
    _start_idx(1, 1)

    for t in range(4):
        pltpu.sync_copy(rows0.at[pl.ds(0, _K), :],
                        acc_sh.at[pl.ds(s * _RPT + t * _K, _K), :])
    pltpu.sync_copy(rows0.at[pl.ds(0, _RPT - 4 * _K), :],
                    acc_sh.at[pl.ds(s * _RPT + 4 * _K, _RPT - 4 * _K), :])

    @pl.when(s == _NS - 1)
    def _zero_rem():
        pltpu.sync_copy(rows0.at[pl.ds(0, _REM), :],
                        acc_sh.at[pl.ds(_NS * _RPT, _REM), :])

    pltpu.sync_copy(dz, deg_sh.at[pl.ds(s * _DPT, _DPT)])

    _wait_idx(0)
    _start_gather(0)
    plsc.subcore_barrier()

    @pl.loop(0, _NFULL, step=2)
    def _chunk(g):
        for b in range(2):
            j = g + b
            nxt = 1 - b

            @pl.when(j + 1 < _NFULL)
            def _prefetch_gather():
                _wait_idx(nxt)
                _start_gather(nxt)

            _wait_gather(b)

            @pl.when(j + 2 < _NFULL)
            def _prefetch_idx():
                _start_idx(j + 2, b)

            d_acc = pltpu.async_copy(rows_b[b], acc_sh.at[didx_b[b]],
                                     ssem, add=True)
            d_deg = pltpu.async_copy(ones_v, deg_sh.at[didx_b[b]],
                                     dsem, add=True)
            d_acc.wait()
            d_deg.wait()

    toff = pl.multiple_of(base + _NFULL * _K, 8)
    pltpu.sync_copy(edge_hbm.at[pl.ds(toff, _TAIL)], sidx_t)
    pltpu.sync_copy(edge_hbm.at[pl.ds(_E + toff, _TAIL)], didx_t)
    pltpu.sync_copy(x_hbm.at[sidx_t], rows_t)
    pltpu.sync_copy(rows_t, acc_sh.at[didx_t], add=True)
    pltpu.sync_copy(ones_v.at[pl.ds(0, _TAIL)], deg_sh.at[didx_t], add=True)

    plsc.subcore_barrier()

    rbase = s * _RPT
    dbase = s * _DPT
    last = _NS * _RPT

    @pl.when(c == 0)
    def _out0():
        pltpu.sync_copy(acc_sh.at[pl.ds(rbase, _RPT), :],
                        acc0_hbm.at[pl.ds(rbase, _RPT), :])
        pltpu.sync_copy(deg_sh.at[pl.ds(dbase, _DPT)],
                        deg0_hbm.at[pl.ds(dbase, _DPT)])

        @pl.when(s == _NS - 1)
        def _rem0():
            pltpu.sync_copy(acc_sh.at[pl.ds(last, _REM), :],
                            acc0_hbm.at[pl.ds(last, _REM), :])

    @pl.when(c == 1)
    def _out1():
        pltpu.sync_copy(acc_sh.at[pl.ds(rbase, _RPT), :],
                        acc1_hbm.at[pl.ds(rbase, _RPT), :])
        pltpu.sync_copy(deg_sh.at[pl.ds(dbase, _DPT)],
                        deg1_hbm.at[pl.ds(dbase, _DPT)])

        @pl.when(s == _NS - 1)
        def _rem1():
            pltpu.sync_copy(acc_sh.at[pl.ds(last, _REM), :],
                            acc1_hbm.at[pl.ds(last, _REM), :])


_sc_call = pl.kernel(
    _sc_body,
    out_type=(
        jax.ShapeDtypeStruct((_N, _D), jnp.float32),
        jax.ShapeDtypeStruct((_N, _D), jnp.float32),
        jax.ShapeDtypeStruct((_DEG_PAD,), jnp.float32),
        jax.ShapeDtypeStruct((_DEG_PAD,), jnp.float32),
    ),
    mesh=plsc.VectorSubcoreMesh(core_axis_name="c", subcore_axis_name="s"),
    scratch_types=[
        pltpu.VMEM_SHARED((_N, _D), jnp.float32),
        pltpu.VMEM_SHARED((_DEG_PAD,), jnp.float32),
        pltpu.VMEM((_K, _D), jnp.float32),
        pltpu.VMEM((_K, _D), jnp.float32),
        pltpu.VMEM((_K,), jnp.int32),
        pltpu.VMEM((_K,), jnp.int32),
        pltpu.VMEM((_K,), jnp.int32),
        pltpu.VMEM((_K,), jnp.int32),
        pltpu.VMEM((_TAIL, _D), jnp.float32),
        pltpu.VMEM((_TAIL,), jnp.int32),
        pltpu.VMEM((_TAIL,), jnp.int32),
        pltpu.VMEM((_K,), jnp.float32),
        pltpu.VMEM((_DPT,), jnp.float32),
        pltpu.SemaphoreType.DMA((2,)),
        pltpu.SemaphoreType.DMA((2,)),
        pltpu.SemaphoreType.DMA,
        pltpu.SemaphoreType.DMA,
    ],
)


def _tc_body(x_ref, a0_ref, a1_ref, d0_ref, d1_ref, o_ref):
    summed = a0_ref[...] + a1_ref[...]
    deg = jnp.maximum(d0_ref[...] + d1_ref[...], 1.0)
    o_ref[:, :_D] = x_ref[...]
    o_ref[:, _D:] = summed / deg


_BLK = 1000


def _tc_call(x, a0, a1, d0, d1):
    return pl.pallas_call(
        _tc_body,
        grid=(_N // _BLK,),
        in_specs=[
            pl.BlockSpec((_BLK, _D), lambda i: (i, 0)),
            pl.BlockSpec((_BLK, _D), lambda i: (i, 0)),
            pl.BlockSpec((_BLK, _D), lambda i: (i, 0)),
            pl.BlockSpec((_BLK, 1), lambda i: (i, 0)),
            pl.BlockSpec((_BLK, 1), lambda i: (i, 0)),
        ],
        out_specs=pl.BlockSpec((_BLK, 2 * _D), lambda i: (i, 0)),
        out_shape=jax.ShapeDtypeStruct((_N, 2 * _D), jnp.float32),
    )(x, a0, a1, d0, d1)


def kernel(x, edge_index):
    x = x.astype(jnp.float32)
    ei = edge_index.astype(jnp.int32).reshape(2 * _E)
    acc0, acc1, deg0, deg1 = _sc_call(x, ei)
    d0 = deg0.reshape(_DEG_PAD, 1)
    d1 = deg1.reshape(_DEG_PAD, 1)
    return _tc_call(x, acc0, acc1, d0, d1)
